# baseline (speedup 1.0000x reference)
import functools

import jax
import jax.numpy as jnp
from jax import lax
from jax.experimental import pallas as pl
from jax.experimental.pallas import tpu as pltpu
from jax.experimental.pallas import tpu_sc as plsc

_CH = 96


def _tc_idx_prep(edge_index, e_pad, n):
    two, e = edge_index.shape
    assert e % 128 == 0 and e_pad % 128 == 0
    er, epr = e // 128, e_pad // 128
    ei3 = edge_index.reshape(two, er, 128)

    def body(ei_ref, src_ref, dst_ref):
        rows = jax.lax.broadcasted_iota(jnp.int32, (epr - er, 128), 0)
        lanes = jax.lax.broadcasted_iota(jnp.int32, (epr - er, 128), 1)
        pad_i = rows * 128 + lanes
        src_ref[...] = jnp.concatenate([ei_ref[0], pad_i & 8191], axis=0)
        dst_ref[...] = jnp.concatenate([ei_ref[1], n + (pad_i & 15)], axis=0)

    src2, dst2 = pl.pallas_call(
        body,
        out_shape=(
            jax.ShapeDtypeStruct((epr, 128), jnp.int32),
            jax.ShapeDtypeStruct((epr, 128), jnp.int32),
        ),
    )(ei3)
    return src2.reshape(e_pad), dst2.reshape(e_pad)


def _tc_stage1(x, W1, b1):
    n, d = x.shape

    def body(x_ref, w_ref, b_ref, o_ref):
        xs = x_ref[...]
        sqn = jnp.sum(xs * xs, axis=1, keepdims=True)
        nrm = jnp.sqrt(jnp.maximum(sqn, 1e-15))
        asinh = jnp.log(nrm + jnp.sqrt(nrm * nrm + 1.0))
        v = (asinh / nrm) * xs
        o_ref[...] = (
            jnp.dot(v, w_ref[...], preferred_element_type=jnp.float32)
            + b_ref[...]
        )

    return pl.pallas_call(
        body, out_shape=jax.ShapeDtypeStruct((n, d), jnp.float32)
    )(x, W1, b1.reshape(1, d))


def _tc_stage3(blob, v1, deg_col, W2, b2):
    n, d = v1.shape

    def body(blob_ref, v1_ref, deg_ref, w_ref, b_ref, v2_ref, rdeg_ref):
        rdeg = 1.0 / (deg_ref[...] + 1.0)
        t = (blob_ref[0, :n, :] + blob_ref[1, :n, :] + v1_ref[...]) * rdeg
        u = t - jnp.mean(t, axis=0, keepdims=True)
        v2_ref[...] = (
            jnp.dot(u, w_ref[...], preferred_element_type=jnp.float32)
            + b_ref[...]
        )
        rdeg_ref[...] = rdeg

    return pl.pallas_call(
        body,
        out_shape=(
            jax.ShapeDtypeStruct((n, d), jnp.float32),
            jax.ShapeDtypeStruct((n, 1), jnp.float32),
        ),
    )(blob, v1, deg_col, W2, b2.reshape(1, d))


def _tc_stage5(blob, v2, rdeg):
    n, d = v2.shape

    def body(blob_ref, v2_ref, rdeg_ref, o_ref):
        t = (blob_ref[0, :n, :] + blob_ref[1, :n, :] + v2_ref[...]) * rdeg_ref[...]
        u = t - jnp.mean(t, axis=0, keepdims=True)
        sqn = jnp.sum(u * u, axis=1, keepdims=True)
        nrm = jnp.sqrt(jnp.maximum(sqn, 1e-15))
        en = jnp.exp(nrm)
        sinh = 0.5 * (en - 1.0 / en)
        o_ref[...] = (sinh / nrm) * u

    return pl.pallas_call(
        body, out_shape=jax.ShapeDtypeStruct((n, d), jnp.float32)
    )(blob, v2, rdeg)


@functools.lru_cache(maxsize=None)
def _make_sc_agg(n_rows, drow, epc, ch, nc, ns, with_deg):
    rows_per_tile = n_rows // ns
    ngrp = epc // 6
    nvec = ch // 16
    assert n_rows % ns == 0 and epc % 6 == 0 and ngrp >= 3 and ch % 16 == 0
    zchunks = []
    off = 0
    while off < rows_per_tile:
        sz = min(ch, rows_per_tile - off)
        zchunks.append((off, sz))
        off += sz
    mesh = plsc.VectorSubcoreMesh(core_axis_name="c", subcore_axis_name="s")

    dgr = -(-n_rows // (128 * 8)) * 8
    out_type = [jax.ShapeDtypeStruct((nc, n_rows, drow), jnp.float32)]
    scratch = [
        [pltpu.VMEM((ch,), jnp.int32)] * 6,
        [pltpu.VMEM((ch,), jnp.int32)] * 6,
        [pltpu.VMEM((ch, drow), jnp.float32)] * 3,
        pltpu.VMEM_SHARED((n_rows, drow), jnp.float32),
        [pltpu.SemaphoreType.DMA] * 6,
        [pltpu.SemaphoreType.DMA] * 6,
        [pltpu.SemaphoreType.DMA] * 3,
    ]
    if with_deg:
        out_type.append(jax.ShapeDtypeStruct((nc, dgr, 128), jnp.float32))
        scratch.append(pltpu.VMEM((dgr, 128), jnp.float32))
        scratch.append(pltpu.VMEM_SHARED((dgr, 128), jnp.float32))
        scratch.append(pltpu.VMEM((1, dgr), jnp.int32))

    def body(tbl, srcf, dstf, zrow, zdeg, iota, out, dego,
             srcv, dstv, bufs, acc, isems, gsems, ssems, degv, dsh, iotv):
        c = lax.axis_index("c")
        s = lax.axis_index("s")
        wid = s * nc + c

        def islice(j):
            return pl.ds(pl.multiple_of((wid * epc + j) * ch, 8), ch)

        def start_idx(j, sl):
            pltpu.async_copy(srcf.at[islice(j)], srcv[sl], isems[sl])
            pltpu.async_copy(dstf.at[islice(j)], dstv[sl], isems[sl])

        def wait_idx(j, sl):
            pltpu.make_async_copy(srcf.at[islice(j)], srcv[sl], isems[sl]).wait()
            pltpu.make_async_copy(dstf.at[islice(j)], dstv[sl], isems[sl]).wait()

        h = ch // 2

        def start_gather(isl, b):
            pltpu.async_copy(
                tbl.at[srcv[isl].at[pl.ds(0, h)]],
                bufs[b].at[pl.ds(0, h)], gsems[2 * b],
            )
            pltpu.async_copy(
                tbl.at[srcv[isl].at[pl.ds(h, h)]],
                bufs[b].at[pl.ds(h, h)], gsems[2 * b + 1],
            )

        def wait_gather(isl, b):
            pltpu.make_async_copy(
                tbl.at[srcv[isl].at[pl.ds(0, h)]],
                bufs[b].at[pl.ds(0, h)], gsems[2 * b],
            ).wait()
            pltpu.make_async_copy(
                tbl.at[srcv[isl].at[pl.ds(h, h)]],
                bufs[b].at[pl.ds(h, h)], gsems[2 * b + 1],
            ).wait()

        def start_scat(isl, b):
            pltpu.async_copy(bufs[b], acc.at[dstv[isl]], ssems[b], add=True)

        def wait_scat(isl, b):
            pltpu.make_async_copy(bufs[b], acc.at[dstv[isl]], ssems[b]).wait()

        def hist(isl):
            if with_deg:
                for kv in range(nvec):
                    vec = dstv[isl][pl.ds(kv * 16, 16)]
                    cnt, last = plsc.scan_count(vec)
                    plsc.addupdate_scatter(
                        degv,
                        [vec >> 7, vec & 127],
                        cnt.astype(jnp.float32),
                        mask=last,
                    )

        for q in range(5):
            start_idx(q, q)
        pltpu.sync_copy(zrow, bufs[0])
        if with_deg:
            pltpu.sync_copy(zdeg, degv)
            pltpu.sync_copy(iota, iotv)

            @pl.when(s == 0)
            def _():
                pltpu.sync_copy(degv, dsh)
        base = s * rows_per_tile
        for off, sz in zchunks:
            pltpu.sync_copy(
                bufs[0].at[pl.ds(0, sz)], acc.at[pl.ds(base + off, sz)]
            )
        wait_idx(0, 0)
        start_gather(0, 0)
        wait_idx(1, 1)
        start_gather(1, 1)
        plsc.subcore_barrier()

        def step(j, r, waits_prev, starts_gather, starts_idx):
            b = r % 3
            wait_gather(r, b)
            start_scat(r, b)
            hist(r)
            if waits_prev:
                wait_scat((r + 5) % 6, (r + 2) % 3)
            if starts_gather:
                wait_idx(j + 2, (r + 2) % 6)
                start_gather((r + 2) % 6, (r + 2) % 3)
            if starts_idx:
                start_idx(j + 5, (r + 5) % 6)

        for r in range(6):
            step(r, r, r > 0, True, True)

        def lbody(g, carry):
            for r in range(6):
                step(6 * g + r, r, True, True, True)
            return carry

        lax.fori_loop(1, ngrp - 1, lbody, 0)
        j0 = epc - 6
        for r in range(6):
            step(j0 + r, r, True, r < 4, r < 1)
        wait_scat(5, 2)
        if with_deg:
            pltpu.sync_copy(degv, dsh.at[iotv.at[0]], add=True)
        plsc.subcore_barrier()
        for off, sz in zchunks:
            pltpu.sync_copy(acc.at[pl.ds(base + off, sz)], bufs[0].at[pl.ds(0, sz)])
            pltpu.sync_copy(
                bufs[0].at[pl.ds(0, sz)], out.at[c, pl.ds(base + off, sz)]
            )
        if with_deg:
            @pl.when(s < dgr // 8)
            def _():
                pltpu.sync_copy(
                    dsh.at[pl.ds(s * 8, 8)], degv.at[pl.ds(0, 8)]
                )
                pltpu.sync_copy(
                    degv.at[pl.ds(0, 8)], dego.at[c, pl.ds(s * 8, 8)]
                )

    if with_deg:
        def k(tbl, srcf, dstf, zrow, zdeg, iota, out, dego,
              srcv, dstv, bufs, acc, isems, gsems, ssems, degv, dsh, iotv):
            body(tbl, srcf, dstf, zrow, zdeg, iota, out, dego,
                 srcv, dstv, bufs, acc, isems, gsems, ssems, degv, dsh, iotv)
    else:
        def k(tbl, srcf, dstf, zrow, out,
              srcv, dstv, bufs, acc, isems, gsems, ssems):
            body(tbl, srcf, dstf, zrow, None, None, out, None,
                 srcv, dstv, bufs, acc, isems, gsems, ssems, None, None, None)

    params = None
    if with_deg:
        params = pltpu.CompilerParams(needs_layout_passes=False)
    return pl.kernel(
        k, out_type=tuple(out_type), mesh=mesh, scratch_types=scratch,
        compiler_params=params,
    )


def kernel(x, edge_index, W1, b1, W2, b2):
    n, d = x.shape
    e = edge_index.shape[1]
    info = plsc.get_sparse_core_info()
    nc, ns = info.num_cores, info.num_subcores
    nw = nc * ns

    n_acc = (n // (8 * ns) + 1) * 8 * ns

    epc = -(-e // (nw * _CH))
    epc += (-epc) % 6
    srcf, dstf = _tc_idx_prep(edge_index, epc * nw * _CH, n)

    dgr = -(-n_acc // (128 * 8)) * 8
    zrow = jnp.zeros((_CH, d), jnp.float32)
    zdeg = jnp.zeros((dgr, 128), jnp.float32)
    iota = jnp.arange(dgr, dtype=jnp.int32).reshape(1, dgr)

    v1 = _tc_stage1(x, W1, b1)
    sc1 = _make_sc_agg(n_acc, d, epc, _CH, nc, ns, True)
    blob1, degp = sc1(v1, srcf, dstf, zrow, zdeg, iota)
    deg_col = (
        (degp[0] + degp[1]).reshape(dgr * 128)[:n].reshape(n, 1)
    )
    v2, rdeg = _tc_stage3(blob1, v1, deg_col, W2, b2)

    sc2 = _make_sc_agg(n_acc, d, epc, _CH, nc, ns, False)
    (blob2,) = sc2(v2, srcf, dstf, zrow)
    return _tc_stage5(blob2, v2, rdeg)


# speedup vs baseline: 1.0083x; 1.0083x over previous
import functools

import jax
import jax.numpy as jnp
from jax import lax
from jax.experimental import pallas as pl
from jax.experimental.pallas import tpu as pltpu
from jax.experimental.pallas import tpu_sc as plsc

_CH = 96


def _tc_idx_prep(edge_index, e_pad, n):
    two, e = edge_index.shape
    assert e % 128 == 0 and e_pad % 128 == 0
    er, epr = e // 128, e_pad // 128
    ei3 = edge_index.reshape(two, er, 128)

    def body(ei_ref, src_ref, dst_ref):
        rows = jax.lax.broadcasted_iota(jnp.int32, (epr - er, 128), 0)
        lanes = jax.lax.broadcasted_iota(jnp.int32, (epr - er, 128), 1)
        pad_i = rows * 128 + lanes
        src_ref[...] = jnp.concatenate([ei_ref[0], pad_i & 8191], axis=0)
        dst_ref[...] = jnp.concatenate([ei_ref[1], n + (pad_i & 15)], axis=0)

    src2, dst2 = pl.pallas_call(
        body,
        out_shape=(
            jax.ShapeDtypeStruct((epr, 128), jnp.int32),
            jax.ShapeDtypeStruct((epr, 128), jnp.int32),
        ),
    )(ei3)
    return src2.reshape(e_pad), dst2.reshape(e_pad)


def _tc_stage1(x, W1, b1):
    n, d = x.shape

    def body(x_ref, w_ref, b_ref, o_ref):
        xs = x_ref[...]
        sqn = jnp.sum(xs * xs, axis=1, keepdims=True)
        nrm = jnp.sqrt(jnp.maximum(sqn, 1e-15))
        asinh = jnp.log(nrm + jnp.sqrt(nrm * nrm + 1.0))
        v = (asinh / nrm) * xs
        o_ref[...] = (
            jnp.dot(v, w_ref[...], preferred_element_type=jnp.float32)
            + b_ref[...]
        )

    return pl.pallas_call(
        body, out_shape=jax.ShapeDtypeStruct((n, d), jnp.float32)
    )(x, W1, b1.reshape(1, d))


def _tc_stage3(blob, v1, deg_col, W2, b2):
    n, d = v1.shape

    def body(blob_ref, v1_ref, deg_ref, w_ref, b_ref, v2_ref, rdeg_ref):
        rdeg = 1.0 / (deg_ref[...] + 1.0)
        t = (blob_ref[0, :n, :] + blob_ref[1, :n, :] + v1_ref[...]) * rdeg
        u = t - jnp.mean(t, axis=0, keepdims=True)
        v2_ref[...] = (
            jnp.dot(u, w_ref[...], preferred_element_type=jnp.float32)
            + b_ref[...]
        )
        rdeg_ref[...] = rdeg

    return pl.pallas_call(
        body,
        out_shape=(
            jax.ShapeDtypeStruct((n, d), jnp.float32),
            jax.ShapeDtypeStruct((n, 1), jnp.float32),
        ),
    )(blob, v1, deg_col, W2, b2.reshape(1, d))


def _tc_stage5(blob, v2, rdeg):
    n, d = v2.shape

    def body(blob_ref, v2_ref, rdeg_ref, o_ref):
        t = (blob_ref[0, :n, :] + blob_ref[1, :n, :] + v2_ref[...]) * rdeg_ref[...]
        u = t - jnp.mean(t, axis=0, keepdims=True)
        sqn = jnp.sum(u * u, axis=1, keepdims=True)
        nrm = jnp.sqrt(jnp.maximum(sqn, 1e-15))
        en = jnp.exp(nrm)
        sinh = 0.5 * (en - 1.0 / en)
        o_ref[...] = (sinh / nrm) * u

    return pl.pallas_call(
        body, out_shape=jax.ShapeDtypeStruct((n, d), jnp.float32)
    )(blob, v2, rdeg)


@functools.lru_cache(maxsize=None)
def _make_sc_agg(n_rows, drow, epc, ch, nc, ns, with_deg):
    rows_per_tile = n_rows // ns
    ngrp = epc // 6
    nvec = ch // 16
    assert n_rows % ns == 0 and epc % 6 == 0 and ngrp >= 3 and ch % 16 == 0
    zchunks = []
    off = 0
    while off < rows_per_tile:
        sz = min(ch, rows_per_tile - off)
        zchunks.append((off, sz))
        off += sz
    mesh = plsc.VectorSubcoreMesh(core_axis_name="c", subcore_axis_name="s")

    dgr = -(-n_rows // (128 * 8)) * 8
    out_type = [jax.ShapeDtypeStruct((nc, n_rows, drow), jnp.float32)]
    scratch = [
        [pltpu.VMEM((ch,), jnp.int32)] * 6,
        [pltpu.VMEM((ch,), jnp.int32)] * 6,
        [pltpu.VMEM((ch, drow), jnp.float32)] * 3,
        pltpu.VMEM_SHARED((n_rows, drow), jnp.float32),
        [pltpu.SemaphoreType.DMA] * 6,
        [pltpu.SemaphoreType.DMA] * 3,
        [pltpu.SemaphoreType.DMA] * 3,
    ]
    if with_deg:
        out_type.append(jax.ShapeDtypeStruct((nc, dgr, 128), jnp.float32))
        scratch.append(pltpu.VMEM((dgr, 128), jnp.float32))
        scratch.append(pltpu.VMEM_SHARED((dgr, 128), jnp.float32))
        scratch.append(pltpu.VMEM((1, dgr), jnp.int32))

    def body(tbl, srcf, dstf, zrow, zdeg, iota, out, dego,
             srcv, dstv, bufs, acc, isems, gsems, ssems, degv, dsh, iotv):
        c = lax.axis_index("c")
        s = lax.axis_index("s")
        wid = s * nc + c

        def islice(j):
            return pl.ds(pl.multiple_of((wid * epc + j) * ch, 8), ch)

        def start_idx(j, sl):
            pltpu.async_copy(srcf.at[islice(j)], srcv[sl], isems[sl])
            pltpu.async_copy(dstf.at[islice(j)], dstv[sl], isems[sl])

        def wait_idx(j, sl):
            pltpu.make_async_copy(srcf.at[islice(j)], srcv[sl], isems[sl]).wait()
            pltpu.make_async_copy(dstf.at[islice(j)], dstv[sl], isems[sl]).wait()

        def start_gather(isl, b):
            pltpu.async_copy(tbl.at[srcv[isl]], bufs[b], gsems[b])

        def wait_gather(isl, b):
            pltpu.make_async_copy(tbl.at[srcv[isl]], bufs[b], gsems[b]).wait()

        def start_scat(isl, b):
            pltpu.async_copy(bufs[b], acc.at[dstv[isl]], ssems[b], add=True)

        def wait_scat(isl, b):
            pltpu.make_async_copy(bufs[b], acc.at[dstv[isl]], ssems[b]).wait()

        def hist(isl):
            if with_deg:
                for kv in range(nvec):
                    vec = dstv[isl][pl.ds(kv * 16, 16)]
                    cnt, last = plsc.scan_count(vec)
                    plsc.addupdate_scatter(
                        degv,
                        [vec >> 7, vec & 127],
                        cnt.astype(jnp.float32),
                        mask=last,
                    )

        for q in range(5):
            start_idx(q, q)
        pltpu.sync_copy(zrow, bufs[0])
        if with_deg:
            pltpu.sync_copy(zdeg, degv)
            pltpu.sync_copy(iota, iotv)

            @pl.when(s == 0)
            def _():
                pltpu.sync_copy(degv, dsh)
        base = s * rows_per_tile
        for off, sz in zchunks:
            pltpu.sync_copy(
                bufs[0].at[pl.ds(0, sz)], acc.at[pl.ds(base + off, sz)]
            )
        wait_idx(0, 0)
        start_gather(0, 0)
        wait_idx(1, 1)
        start_gather(1, 1)
        plsc.subcore_barrier()

        def step(j, r, waits_prev, starts_gather, starts_idx):
            b = r % 3
            wait_gather(r, b)
            start_scat(r, b)
            hist(r)
            if waits_prev:
                wait_scat((r + 5) % 6, (r + 2) % 3)
            if starts_gather:
                wait_idx(j + 2, (r + 2) % 6)
                start_gather((r + 2) % 6, (r + 2) % 3)
            if starts_idx:
                start_idx(j + 5, (r + 5) % 6)

        for r in range(6):
            step(r, r, r > 0, True, True)

        def lbody(g, carry):
            for r in range(6):
                step(6 * g + r, r, True, True, True)
            return carry

        lax.fori_loop(1, ngrp - 1, lbody, 0)
        j0 = epc - 6
        for r in range(6):
            step(j0 + r, r, True, r < 4, r < 1)
        wait_scat(5, 2)
        if with_deg:
            pltpu.sync_copy(degv, dsh.at[iotv.at[0]], add=True)
        plsc.subcore_barrier()
        for off, sz in zchunks:
            pltpu.sync_copy(acc.at[pl.ds(base + off, sz)], bufs[0].at[pl.ds(0, sz)])
            pltpu.sync_copy(
                bufs[0].at[pl.ds(0, sz)], out.at[c, pl.ds(base + off, sz)]
            )
        if with_deg:
            @pl.when(s < dgr // 8)
            def _():
                pltpu.sync_copy(
                    dsh.at[pl.ds(s * 8, 8)], degv.at[pl.ds(0, 8)]
                )
                pltpu.sync_copy(
                    degv.at[pl.ds(0, 8)], dego.at[c, pl.ds(s * 8, 8)]
                )

    if with_deg:
        def k(tbl, srcf, dstf, zrow, zdeg, iota, out, dego,
              srcv, dstv, bufs, acc, isems, gsems, ssems, degv, dsh, iotv):
            body(tbl, srcf, dstf, zrow, zdeg, iota, out, dego,
                 srcv, dstv, bufs, acc, isems, gsems, ssems, degv, dsh, iotv)
    else:
        def k(tbl, srcf, dstf, zrow, out,
              srcv, dstv, bufs, acc, isems, gsems, ssems):
            body(tbl, srcf, dstf, zrow, None, None, out, None,
                 srcv, dstv, bufs, acc, isems, gsems, ssems, None, None, None)

    params = None
    if with_deg:
        params = pltpu.CompilerParams(needs_layout_passes=False)
    return pl.kernel(
        k, out_type=tuple(out_type), mesh=mesh, scratch_types=scratch,
        compiler_params=params,
    )


def kernel(x, edge_index, W1, b1, W2, b2):
    n, d = x.shape
    e = edge_index.shape[1]
    info = plsc.get_sparse_core_info()
    nc, ns = info.num_cores, info.num_subcores
    nw = nc * ns

    n_acc = (n // (8 * ns) + 1) * 8 * ns

    epc = -(-e // (nw * _CH))
    epc += (-epc) % 6
    srcf, dstf = _tc_idx_prep(edge_index, epc * nw * _CH, n)

    dgr = -(-n_acc // (128 * 8)) * 8
    zrow = jnp.zeros((_CH, d), jnp.float32)
    zdeg = jnp.zeros((dgr, 128), jnp.float32)
    iota = jnp.arange(dgr, dtype=jnp.int32).reshape(1, dgr)

    v1 = _tc_stage1(x, W1, b1)
    sc1 = _make_sc_agg(n_acc, d, epc, _CH, nc, ns, True)
    blob1, degp = sc1(v1, srcf, dstf, zrow, zdeg, iota)
    deg_col = (
        (degp[0] + degp[1]).reshape(dgr * 128)[:n].reshape(n, 1)
    )
    v2, rdeg = _tc_stage3(blob1, v1, deg_col, W2, b2)

    sc2 = _make_sc_agg(n_acc, d, epc, _CH, nc, ns, False)
    (blob2,) = sc2(v2, srcf, dstf, zrow)
    return _tc_stage5(blob2, v2, rdeg)
